# baseline (speedup 1.0000x reference)
import dataclasses
import functools

import jax
import jax.numpy as jnp
from jax import lax
from jax.experimental import pallas as pl
from jax.experimental.pallas import tpu as pltpu
from jax.experimental.pallas import tpu_sc as plsc

_N = 10000
_NP = 10240
_E = 320000
_EP = 327680
_D = 128
_NG = 64
_EPS = 1e-5
_NCORE = 2
_NSUB = 16
_RPT = _NP // _NSUB
_EBLK = 128
_ET = _EP // (_NCORE * _NSUB)
_NBLK = _ET // _EBLK
_BM = 256
_GRID = _NP // _BM
_IRING = 5
_PREC = jax.lax.Precision.HIGHEST
_CP = dataclasses.replace(pltpu.CompilerParams(), needs_layout_passes=False)


def _sc_degree(dstp):
    mesh = plsc.VectorSubcoreMesh(core_axis_name="c", subcore_axis_name="s")

    @functools.partial(
        pl.kernel,
        mesh=mesh,
        out_type=jax.ShapeDtypeStruct((_NCORE, _NP), jnp.float32),
        scratch_types=[
            pltpu.VMEM((_ET,), jnp.int32),
            pltpu.VMEM((_NP,), jnp.float32),
            pltpu.VMEM((_NSUB, _RPT), jnp.float32),
            pltpu.VMEM_SHARED((_NSUB, _NP), jnp.float32),
        ],
        compiler_params=_CP,
    )
    def deg_kernel(dst_hbm, out_hbm, ebuf, hist, rbuf, slots):
        c = lax.axis_index("c")
        s = lax.axis_index("s")

        @pl.loop(0, _NP // 16)
        def _(j):
            hist[pl.ds(j * 16, 16)] = jnp.zeros((16,), jnp.float32)

        pltpu.sync_copy(dst_hbm.at[pl.ds(c * (_EP // 2) + s * _ET, _ET)], ebuf)
        ones = jnp.ones((16,), jnp.float32)

        @pl.loop(0, _ET // 16)
        def _(j):
            plsc.addupdate_scatter(hist, [ebuf[pl.ds(j * 16, 16)]], ones)

        pltpu.sync_copy(hist, slots.at[s])
        plsc.subcore_barrier()
        row0 = s * _RPT
        pltpu.sync_copy(slots.at[:, pl.ds(row0, _RPT)], rbuf)

        @pl.loop(0, _RPT // 16)
        def _(j):
            tot = rbuf[0, pl.ds(j * 16, 16)]
            for k in range(1, _NSUB):
                tot = tot + rbuf[k, pl.ds(j * 16, 16)]
            hist[pl.ds(j * 16, 16)] = tot

        pltpu.sync_copy(hist.at[pl.ds(0, _RPT)], out_hbm.at[c, pl.ds(row0, _RPT)])

    return deg_kernel(dstp)


def _sc_aggregate(h, eidx, zerosD):
    mesh = plsc.VectorSubcoreMesh(core_axis_name="c", subcore_axis_name="s")

    @functools.partial(
        pl.kernel,
        mesh=mesh,
        out_type=jax.ShapeDtypeStruct((_NCORE, _NP, _D), jnp.float32),
        scratch_types=[
            pltpu.VMEM((2, _EBLK), jnp.int32),
            pltpu.VMEM((2, _EBLK), jnp.int32),
            pltpu.VMEM((_EBLK, _D), jnp.float32),
            pltpu.VMEM((_EBLK, _D), jnp.float32),
            pltpu.VMEM_SHARED((_NP, _D), jnp.float32),
            pltpu.SemaphoreType.DMA,
            pltpu.SemaphoreType.DMA,
        ],
    )
    def agg_kernel(h_hbm, e_hbm, z_hbm, out_hbm, idx0, idx1, rows0, rows1,
                   acc, ss0, ss1):
        idx = (idx0, idx1)
        rows = (rows0, rows1)
        ss = (ss0, ss1)
        c = lax.axis_index("c")
        s = lax.axis_index("s")
        row0 = s * _RPT
        gb0 = (c * _NSUB + s) * _NBLK
        pltpu.sync_copy(z_hbm.at[pl.ds(row0, _RPT)], acc.at[pl.ds(row0, _RPT)])
        plsc.subcore_barrier()

        @pl.loop(0, _NBLK)
        def _(blk):
            pltpu.sync_copy(e_hbm.at[gb0 + blk], idx0)
            pltpu.sync_copy(h_hbm.at[idx0.at[0]], rows0)
            pltpu.sync_copy(rows0, acc.at[idx0.at[1]], add=True)

        plsc.subcore_barrier()
        pltpu.sync_copy(acc.at[pl.ds(row0, _RPT)],
                        out_hbm.at[c, pl.ds(row0, _RPT)])

    return agg_kernel(h, eidx, zerosD)


def _dinv_of(dg_ref):
    d = dg_ref[0] + dg_ref[1] + 1.0
    return lax.rsqrt(d)


_DEG_SPEC = pl.BlockSpec((_NCORE, _BM, 1), lambda i: (0, i, 0))


def _tc_prep1(x_pad, W1, degp):
    def body(x_ref, w_ref, dg_ref, o_ref):
        h = jnp.dot(x_ref[...], w_ref[...],
                    preferred_element_type=jnp.float32)
        o_ref[...] = h * _dinv_of(dg_ref)

    return pl.pallas_call(
        body,
        grid=(_GRID,),
        in_specs=[
            pl.BlockSpec((_BM, _D), lambda i: (i, 0)),
            pl.BlockSpec((_D, _D), lambda i: (0, 0)),
            _DEG_SPEC,
        ],
        out_specs=pl.BlockSpec((_BM, _D), lambda i: (i, 0)),
        out_shape=jax.ShapeDtypeStruct((_NP, _D), jnp.float32),
    )(x_pad, W1, degp)


def _tc_stats(acc, hp, degp, bias):
    def body(a_ref, h_ref, dg_ref, b_ref, z_ref, st_ref):
        i = pl.program_id(0)
        zz = (a_ref[0] + a_ref[1] + h_ref[...]) * _dinv_of(dg_ref) + b_ref[...]
        z_ref[...] = zz
        rid = lax.broadcasted_iota(jnp.int32, (_BM, 1), 0) + i * _BM
        zm = zz * (rid < _N).astype(jnp.float32)

        @pl.when(i == 0)
        def _():
            st_ref[...] = jnp.zeros_like(st_ref)

        st_ref[...] += jnp.concatenate(
            [jnp.sum(zm, axis=0, keepdims=True),
             jnp.sum(zz * zm, axis=0, keepdims=True)], axis=0)

    return pl.pallas_call(
        body,
        grid=(_GRID,),
        in_specs=[
            pl.BlockSpec((_NCORE, _BM, _D), lambda i: (0, i, 0)),
            pl.BlockSpec((_BM, _D), lambda i: (i, 0)),
            _DEG_SPEC,
            pl.BlockSpec((1, _D), lambda i: (0, 0)),
        ],
        out_specs=[pl.BlockSpec((_BM, _D), lambda i: (i, 0)),
                   pl.BlockSpec((2, _D), lambda i: (0, 0))],
        out_shape=[jax.ShapeDtypeStruct((_NP, _D), jnp.float32),
                   jax.ShapeDtypeStruct((2, _D), jnp.float32)],
    )(acc, hp, degp, bias)


def _bn_relu(z, st_ref, g_ref, be_ref):
    mu = st_ref[0:1] / _N
    var = st_ref[1:2] / _N - mu * mu
    inv = lax.rsqrt(var + _EPS)
    return jnp.maximum((z - mu) * inv * g_ref[...] + be_ref[...], 0.0)


def _tc_norm_mm(z, st, gamma, beta, W2, degp):
    def body(z_ref, st_ref, g_ref, be_ref, w_ref, dg_ref, o_ref):
        y = _bn_relu(z_ref[...], st_ref, g_ref, be_ref)
        h = jnp.dot(y, w_ref[...],
                    preferred_element_type=jnp.float32)
        o_ref[...] = h * _dinv_of(dg_ref)

    return pl.pallas_call(
        body,
        grid=(_GRID,),
        in_specs=[
            pl.BlockSpec((_BM, _D), lambda i: (i, 0)),
            pl.BlockSpec((2, _D), lambda i: (0, 0)),
            pl.BlockSpec((1, _D), lambda i: (0, 0)),
            pl.BlockSpec((1, _D), lambda i: (0, 0)),
            pl.BlockSpec((_D, _D), lambda i: (0, 0)),
            _DEG_SPEC,
        ],
        out_specs=pl.BlockSpec((_BM, _D), lambda i: (i, 0)),
        out_shape=jax.ShapeDtypeStruct((_NP, _D), jnp.float32),
    )(z, st, gamma, beta, W2, degp)


def _tc_pool(z, st, gamma, beta, batchcol):
    def body(z_ref, st_ref, g_ref, be_ref, b_ref, ps_ref, cm_ref):
        i = pl.program_id(0)
        y = _bn_relu(z_ref[...], st_ref, g_ref, be_ref)
        gids = lax.broadcasted_iota(jnp.int32, (_BM, _NG), 1).astype(jnp.float32)
        oh = (b_ref[...] == gids).astype(jnp.float32)
        ps_blk = lax.dot_general(oh, y, (((0,), (0,)), ((), ())),
                                 precision=_PREC,
                                 preferred_element_type=jnp.float32)
        cm_blk = lax.dot_general(oh, jnp.ones((_BM, _D), jnp.float32),
                                 (((0,), (0,)), ((), ())),
                                 precision=_PREC,
                                 preferred_element_type=jnp.float32)

        @pl.when(i == 0)
        def _():
            ps_ref[...] = jnp.zeros_like(ps_ref)
            cm_ref[...] = jnp.zeros_like(cm_ref)

        ps_ref[...] += ps_blk
        cm_ref[...] += cm_blk

    return pl.pallas_call(
        body,
        grid=(_GRID,),
        in_specs=[
            pl.BlockSpec((_BM, _D), lambda i: (i, 0)),
            pl.BlockSpec((2, _D), lambda i: (0, 0)),
            pl.BlockSpec((1, _D), lambda i: (0, 0)),
            pl.BlockSpec((1, _D), lambda i: (0, 0)),
            pl.BlockSpec((_BM, 1), lambda i: (i, 0)),
        ],
        out_specs=[pl.BlockSpec((_NG, _D), lambda i: (0, 0)),
                   pl.BlockSpec((_NG, _D), lambda i: (0, 0))],
        out_shape=[jax.ShapeDtypeStruct((_NG, _D), jnp.float32),
                   jax.ShapeDtypeStruct((_NG, _D), jnp.float32)],
    )(z, st, gamma, beta, batchcol)


def _tc_head(ps, cm, Wf, bfb, gfb, befb, wlT, blb):
    def body(ps_ref, cm_ref, wf_ref, bf_ref, gf_ref, bef_ref, wl_ref, bl_ref,
             o_ref):
        pooled = ps_ref[...] / jnp.maximum(cm_ref[...], 1.0)
        t = jnp.dot(pooled, wf_ref[...],
                    preferred_element_type=jnp.float32)
        t = t + bf_ref[...]
        mu = jnp.mean(t, axis=0, keepdims=True)
        var = jnp.mean(t * t, axis=0, keepdims=True) - mu * mu
        f = jnp.maximum((t - mu) * lax.rsqrt(var + _EPS) * gf_ref[...]
                        + bef_ref[...], 0.0)
        o = jnp.sum(f * wl_ref[...], axis=1, keepdims=True) + bl_ref[...]
        o_ref[...] = jnp.maximum(o, 0.0)

    return pl.pallas_call(
        body,
        out_shape=jax.ShapeDtypeStruct((_NG, 1), jnp.float32),
    )(ps, cm, Wf, bfb, gfb, befb, wlT, blb)


def kernel(x, edge_index, batch, W1, b1, g1, be1, W2, b2, g2, be2,
           Wf, bf, gf, bef, Wl, bl):
    f32 = jnp.float32
    x_pad = jnp.pad(x, ((0, _NP - _N), (0, 0)))
    ei = edge_index.astype(jnp.int32)
    srcp = jnp.pad(ei[0], (0, _EP - _E), constant_values=_N)
    dstp = jnp.pad(ei[1], (0, _EP - _E), constant_values=_N)
    eidx = jnp.stack([srcp.reshape(-1, _EBLK), dstp.reshape(-1, _EBLK)], axis=1)
    batchcol = jnp.pad(batch.astype(jnp.int32), (0, _NP - _N),
                       constant_values=_NG).astype(f32).reshape(_NP, 1)
    zerosD = jnp.zeros((_NP, _D), f32)

    degp = _sc_degree(dstp).reshape(_NCORE, _NP, 1)
    h1p = _tc_prep1(x_pad, W1, degp)
    acc1 = _sc_aggregate(h1p, eidx, zerosD)
    z1, st1 = _tc_stats(acc1, h1p, degp, b1.reshape(1, _D))
    h2p = _tc_norm_mm(z1, st1, g1.reshape(1, _D), be1.reshape(1, _D), W2, degp)
    acc2 = _sc_aggregate(h2p, eidx, zerosD)
    z2, st2 = _tc_stats(acc2, h2p, degp, b2.reshape(1, _D))
    ps, cm = _tc_pool(z2, st2, g2.reshape(1, _D), be2.reshape(1, _D), batchcol)
    return _tc_head(ps, cm, Wf, bf.reshape(1, _D), gf.reshape(1, _D),
                    bef.reshape(1, _D), Wl.reshape(1, _D), bl.reshape(1, 1))


# speedup vs baseline: 1.3972x; 1.3972x over previous
import dataclasses
import functools

import jax
import jax.numpy as jnp
from jax import lax
from jax.experimental import pallas as pl
from jax.experimental.pallas import tpu as pltpu
from jax.experimental.pallas import tpu_sc as plsc

_N = 10000
_NP = 10240
_E = 320000
_EP = 323584
_D = 128
_NG = 64
_EPS = 1e-5
_NCORE = 2
_NSUB = 16
_RPT = _NP // _NSUB
_EBLK = 128
_ET = _EP // (_NCORE * _NSUB)
_NBLK = _ET // _EBLK
_BM = 256
_GRID = _NP // _BM
_PREC = jax.lax.Precision.HIGHEST
_CP = dataclasses.replace(pltpu.CompilerParams(), needs_layout_passes=False)


def _sc_degree(dstp):
    mesh = plsc.VectorSubcoreMesh(core_axis_name="c", subcore_axis_name="s")

    @functools.partial(
        pl.kernel,
        mesh=mesh,
        out_type=jax.ShapeDtypeStruct((_NCORE, _NP), jnp.float32),
        scratch_types=[
            pltpu.VMEM((_ET,), jnp.int32),
            pltpu.VMEM((_NP,), jnp.float32),
            pltpu.VMEM((_NSUB, _RPT), jnp.float32),
            pltpu.VMEM_SHARED((_NSUB, _NP), jnp.float32),
        ],
        compiler_params=_CP,
    )
    def deg_kernel(dst_hbm, out_hbm, ebuf, hist, rbuf, slots):
        c = lax.axis_index("c")
        s = lax.axis_index("s")

        @pl.loop(0, _NP // 16)
        def _(j):
            hist[pl.ds(j * 16, 16)] = jnp.zeros((16,), jnp.float32)

        pltpu.sync_copy(dst_hbm.at[pl.ds(c * (_EP // 2) + s * _ET, _ET)], ebuf)
        ones = jnp.ones((16,), jnp.float32)

        @pl.loop(0, _ET // 16)
        def _(j):
            plsc.addupdate_scatter(hist, [ebuf[pl.ds(j * 16, 16)]], ones)

        pltpu.sync_copy(hist, slots.at[s])
        plsc.subcore_barrier()
        row0 = s * _RPT
        pltpu.sync_copy(slots.at[:, pl.ds(row0, _RPT)], rbuf)

        @pl.loop(0, _RPT // 16)
        def _(j):
            tot = rbuf[0, pl.ds(j * 16, 16)]
            for k in range(1, _NSUB):
                tot = tot + rbuf[k, pl.ds(j * 16, 16)]
            hist[pl.ds(j * 16, 16)] = tot

        pltpu.sync_copy(hist.at[pl.ds(0, _RPT)], out_hbm.at[c, pl.ds(row0, _RPT)])

    return deg_kernel(dstp)


def _sc_aggregate(h, srcp, dstp, zerosD):
    mesh = plsc.VectorSubcoreMesh(core_axis_name="c", subcore_axis_name="s")

    @functools.partial(
        pl.kernel,
        mesh=mesh,
        out_type=jax.ShapeDtypeStruct((_NCORE, _NP, _D), jnp.float32),
        scratch_types=[
            pltpu.VMEM((_EBLK,), jnp.int32),
            pltpu.VMEM((_EBLK,), jnp.int32),
            pltpu.VMEM((_EBLK, _D), jnp.float32),
            pltpu.VMEM_SHARED((_NP, _D), jnp.float32),
        ],
    )
    def agg_kernel(h_hbm, src_hbm, dst_hbm, z_hbm, out_hbm, sidx, didx, rows, acc):
        c = lax.axis_index("c")
        s = lax.axis_index("s")
        row0 = s * _RPT
        pltpu.sync_copy(z_hbm.at[pl.ds(row0, _RPT)], acc.at[pl.ds(row0, _RPT)])
        plsc.subcore_barrier()
        ebase = c * (_EP // 2) + s * _ET

        @pl.loop(0, _NBLK)
        def _(i):
            e0 = ebase + i * _EBLK
            pltpu.sync_copy(src_hbm.at[pl.ds(e0, _EBLK)], sidx)
            pltpu.sync_copy(dst_hbm.at[pl.ds(e0, _EBLK)], didx)
            pltpu.sync_copy(h_hbm.at[sidx], rows)
            pltpu.sync_copy(rows, acc.at[didx], add=True)

        plsc.subcore_barrier()
        pltpu.sync_copy(acc.at[pl.ds(row0, _RPT)],
                        out_hbm.at[c, pl.ds(row0, _RPT)])

    return agg_kernel(h, srcp, dstp, zerosD)


def _dinv_of(dg_ref):
    d = dg_ref[0] + dg_ref[1] + 1.0
    return lax.rsqrt(d)


_DEG_SPEC = pl.BlockSpec((_NCORE, _BM, 1), lambda i: (0, i, 0))


def _tc_prep1(x_pad, W1, degp):
    def body(x_ref, w_ref, dg_ref, o_ref):
        h = jnp.dot(x_ref[...], w_ref[...],
                    preferred_element_type=jnp.float32)
        o_ref[...] = h * _dinv_of(dg_ref)

    return pl.pallas_call(
        body,
        grid=(_GRID,),
        in_specs=[
            pl.BlockSpec((_BM, _D), lambda i: (i, 0)),
            pl.BlockSpec((_D, _D), lambda i: (0, 0)),
            _DEG_SPEC,
        ],
        out_specs=pl.BlockSpec((_BM, _D), lambda i: (i, 0)),
        out_shape=jax.ShapeDtypeStruct((_NP, _D), jnp.float32),
    )(x_pad, W1, degp)


def _tc_stats(acc, hp, degp, bias):
    def body(a_ref, h_ref, dg_ref, b_ref, z_ref, st_ref):
        i = pl.program_id(0)
        zz = (a_ref[0] + a_ref[1] + h_ref[...]) * _dinv_of(dg_ref) + b_ref[...]
        z_ref[...] = zz
        rid = lax.broadcasted_iota(jnp.int32, (_BM, 1), 0) + i * _BM
        zm = zz * (rid < _N).astype(jnp.float32)

        @pl.when(i == 0)
        def _():
            st_ref[...] = jnp.zeros_like(st_ref)

        st_ref[...] += jnp.concatenate(
            [jnp.sum(zm, axis=0, keepdims=True),
             jnp.sum(zz * zm, axis=0, keepdims=True)], axis=0)

    return pl.pallas_call(
        body,
        grid=(_GRID,),
        in_specs=[
            pl.BlockSpec((_NCORE, _BM, _D), lambda i: (0, i, 0)),
            pl.BlockSpec((_BM, _D), lambda i: (i, 0)),
            _DEG_SPEC,
            pl.BlockSpec((1, _D), lambda i: (0, 0)),
        ],
        out_specs=[pl.BlockSpec((_BM, _D), lambda i: (i, 0)),
                   pl.BlockSpec((2, _D), lambda i: (0, 0))],
        out_shape=[jax.ShapeDtypeStruct((_NP, _D), jnp.float32),
                   jax.ShapeDtypeStruct((2, _D), jnp.float32)],
    )(acc, hp, degp, bias)


def _bn_relu(z, st_ref, g_ref, be_ref):
    mu = st_ref[0:1] / _N
    var = st_ref[1:2] / _N - mu * mu
    inv = lax.rsqrt(var + _EPS)
    return jnp.maximum((z - mu) * inv * g_ref[...] + be_ref[...], 0.0)


def _tc_norm_mm(z, st, gamma, beta, W2, degp):
    def body(z_ref, st_ref, g_ref, be_ref, w_ref, dg_ref, o_ref):
        y = _bn_relu(z_ref[...], st_ref, g_ref, be_ref)
        h = jnp.dot(y, w_ref[...],
                    preferred_element_type=jnp.float32)
        o_ref[...] = h * _dinv_of(dg_ref)

    return pl.pallas_call(
        body,
        grid=(_GRID,),
        in_specs=[
            pl.BlockSpec((_BM, _D), lambda i: (i, 0)),
            pl.BlockSpec((2, _D), lambda i: (0, 0)),
            pl.BlockSpec((1, _D), lambda i: (0, 0)),
            pl.BlockSpec((1, _D), lambda i: (0, 0)),
            pl.BlockSpec((_D, _D), lambda i: (0, 0)),
            _DEG_SPEC,
        ],
        out_specs=pl.BlockSpec((_BM, _D), lambda i: (i, 0)),
        out_shape=jax.ShapeDtypeStruct((_NP, _D), jnp.float32),
    )(z, st, gamma, beta, W2, degp)


def _tc_pool(z, st, gamma, beta, batchcol):
    def body(z_ref, st_ref, g_ref, be_ref, b_ref, ps_ref, cm_ref):
        i = pl.program_id(0)
        y = _bn_relu(z_ref[...], st_ref, g_ref, be_ref)
        gids = lax.broadcasted_iota(jnp.int32, (_BM, _NG), 1).astype(jnp.float32)
        oh = (b_ref[...] == gids).astype(jnp.float32)
        ps_blk = lax.dot_general(oh, y, (((0,), (0,)), ((), ())),
                                 precision=_PREC,
                                 preferred_element_type=jnp.float32)
        cm_blk = lax.dot_general(oh, jnp.ones((_BM, _D), jnp.float32),
                                 (((0,), (0,)), ((), ())),
                                 precision=_PREC,
                                 preferred_element_type=jnp.float32)

        @pl.when(i == 0)
        def _():
            ps_ref[...] = jnp.zeros_like(ps_ref)
            cm_ref[...] = jnp.zeros_like(cm_ref)

        ps_ref[...] += ps_blk
        cm_ref[...] += cm_blk

    return pl.pallas_call(
        body,
        grid=(_GRID,),
        in_specs=[
            pl.BlockSpec((_BM, _D), lambda i: (i, 0)),
            pl.BlockSpec((2, _D), lambda i: (0, 0)),
            pl.BlockSpec((1, _D), lambda i: (0, 0)),
            pl.BlockSpec((1, _D), lambda i: (0, 0)),
            pl.BlockSpec((_BM, 1), lambda i: (i, 0)),
        ],
        out_specs=[pl.BlockSpec((_NG, _D), lambda i: (0, 0)),
                   pl.BlockSpec((_NG, _D), lambda i: (0, 0))],
        out_shape=[jax.ShapeDtypeStruct((_NG, _D), jnp.float32),
                   jax.ShapeDtypeStruct((_NG, _D), jnp.float32)],
    )(z, st, gamma, beta, batchcol)


def _tc_head(ps, cm, Wf, bfb, gfb, befb, wlT, blb):
    def body(ps_ref, cm_ref, wf_ref, bf_ref, gf_ref, bef_ref, wl_ref, bl_ref,
             o_ref):
        pooled = ps_ref[...] / jnp.maximum(cm_ref[...], 1.0)
        t = jnp.dot(pooled, wf_ref[...],
                    preferred_element_type=jnp.float32)
        t = t + bf_ref[...]
        mu = jnp.mean(t, axis=0, keepdims=True)
        var = jnp.mean(t * t, axis=0, keepdims=True) - mu * mu
        f = jnp.maximum((t - mu) * lax.rsqrt(var + _EPS) * gf_ref[...]
                        + bef_ref[...], 0.0)
        o = jnp.sum(f * wl_ref[...], axis=1, keepdims=True) + bl_ref[...]
        o_ref[...] = jnp.maximum(o, 0.0)

    return pl.pallas_call(
        body,
        out_shape=jax.ShapeDtypeStruct((_NG, 1), jnp.float32),
    )(ps, cm, Wf, bfb, gfb, befb, wlT, blb)


def kernel(x, edge_index, batch, W1, b1, g1, be1, W2, b2, g2, be2,
           Wf, bf, gf, bef, Wl, bl):
    f32 = jnp.float32
    x_pad = jnp.pad(x, ((0, _NP - _N), (0, 0)))
    ei = edge_index.astype(jnp.int32)
    srcp = jnp.pad(ei[0], (0, _EP - _E), constant_values=_N)
    dstp = jnp.pad(ei[1], (0, _EP - _E), constant_values=_N)
    batchcol = jnp.pad(batch.astype(jnp.int32), (0, _NP - _N),
                       constant_values=_NG).astype(f32).reshape(_NP, 1)
    zerosD = jnp.zeros((_NP, _D), f32)

    degp = _sc_degree(dstp).reshape(_NCORE, _NP, 1)
    h1p = _tc_prep1(x_pad, W1, degp)
    acc1 = _sc_aggregate(h1p, srcp, dstp, zerosD)
    z1, st1 = _tc_stats(acc1, h1p, degp, b1.reshape(1, _D))
    h2p = _tc_norm_mm(z1, st1, g1.reshape(1, _D), be1.reshape(1, _D), W2, degp)
    acc2 = _sc_aggregate(h2p, srcp, dstp, zerosD)
    z2, st2 = _tc_stats(acc2, h2p, degp, b2.reshape(1, _D))
    ps, cm = _tc_pool(z2, st2, g2.reshape(1, _D), be2.reshape(1, _D), batchcol)
    return _tc_head(ps, cm, Wf, bf.reshape(1, _D), gf.reshape(1, _D),
                    bef.reshape(1, _D), Wl.reshape(1, _D), bl.reshape(1, 1))
